# baseline (speedup 1.0000x reference)
import functools

import jax
import jax.numpy as jnp
from jax import lax
from jax.experimental import pallas as pl
from jax.experimental.pallas import tpu as pltpu
from jax.experimental.pallas import tpu_sc as plsc

N = 16384
V = 1000000
D = 64
E = 4

NC = 2
NS = 16
L = 16
NW = NC * NS
BPW = N // NW
GPB = 8
BLK = GPB * L
NBLK = BPW // BLK


def _sc_body(nodes_hbm, emb_hbm, wb_hbm, ew_hbm, out_hbm,
             idx_v, rows_v, wb_v, ew_v, sco_v, res_v, sem):
    wid = lax.axis_index("s") * NC + lax.axis_index("c")
    base = wid * BPW

    pltpu.sync_copy(nodes_hbm.at[pl.ds(base, BPW)], idx_v)
    gather = pltpu.async_copy(emb_hbm.at[idx_v], rows_v, sem)
    pltpu.sync_copy(wb_hbm, wb_v)
    pltpu.sync_copy(ew_hbm, ew_v)
    gather.wait()

    lanes = lax.iota(jnp.int32, L)
    zero = jnp.zeros((L,), jnp.float32)

    def block_body(b, carry):
        tok0 = b * BLK
        toks = [tok0 + g * L + lanes for g in range(GPB)]

        def d_body(d, accs):
            w = [wb_v[e, d] for e in range(E)]
            dvec = jnp.full((L,), 0, jnp.int32) + d
            new = []
            for g in range(GPB):
                col = plsc.load_gather(rows_v, [toks[g], dvec])
                prev = accs[g]
                new.append(tuple(prev[e] + col * w[e] for e in range(E)))
            return tuple(new)

        init = tuple(tuple(zero for _ in range(E)) for _ in range(GPB))
        accs = lax.fori_loop(0, D, d_body, init)

        for g in range(GPB):
            s0, s1, s2, s3 = accs[g]
            m = jnp.maximum(jnp.maximum(s0, s1), jnp.maximum(s2, s3))
            e0 = jnp.exp(s0 - m)
            e1 = jnp.exp(s1 - m)
            e2 = jnp.exp(s2 - m)
            e3 = jnp.exp(s3 - m)
            r = 1.0 / (e0 + e1 + e2 + e3)
            p0, p1, p2, p3 = e0 * r, e1 * r, e2 * r, e3 * r
            c0 = (s0 >= s1) & (s0 >= s2) & (s0 >= s3)
            c1 = (s1 >= s2) & (s1 >= s3)
            c2 = s2 >= s3
            two = jnp.full((L,), 2, jnp.int32)
            three = jnp.full((L,), 3, jnp.int32)
            one = jnp.full((L,), 1, jnp.int32)
            zi = jnp.full((L,), 0, jnp.int32)
            choice = jnp.where(c0, zi, jnp.where(c1, one, jnp.where(c2, two, three)))
            c4 = choice * 4
            ps = (p0, p1, p2, p3)
            res = zero
            for j in range(E):
                wj = plsc.load_gather(ew_v, [c4 + jnp.full((L,), j, jnp.int32)])
                res = res + wj * ps[j]
            off = tok0 + g * L
            sco_v[0, pl.ds(off, L)] = p0
            sco_v[1, pl.ds(off, L)] = p1
            sco_v[2, pl.ds(off, L)] = p2
            sco_v[3, pl.ds(off, L)] = p3
            res_v[pl.ds(off, L)] = res
        return carry

    lax.fori_loop(0, NBLK, block_body, 0)

    for e in range(E):
        pltpu.sync_copy(sco_v.at[e], out_hbm.at[pl.ds(e * N + base, BPW)])
    pltpu.sync_copy(res_v, out_hbm.at[pl.ds(E * N + base, BPW)])


@jax.jit
def kernel(nodes, emb_table, W, expert_w):
    wb = jnp.broadcast_to(W[:, :, None], (E, D, L)).astype(jnp.float32)
    ewf = expert_w.reshape(-1).astype(jnp.float32)
    nodes32 = nodes.astype(jnp.int32)
    mesh = plsc.VectorSubcoreMesh(core_axis_name="c", subcore_axis_name="s")
    f = pl.kernel(
        _sc_body,
        out_type=jax.ShapeDtypeStruct((E * N + N,), jnp.float32),
        mesh=mesh,
        compiler_params=pltpu.CompilerParams(
            needs_layout_passes=False, use_tc_tiling_on_sc=False),
        scratch_types=[
            pltpu.VMEM((BPW,), jnp.int32),
            pltpu.VMEM((BPW, D), jnp.float32),
            pltpu.VMEM((E, D, L), jnp.float32),
            pltpu.VMEM((E * E,), jnp.float32),
            pltpu.VMEM((E, BPW), jnp.float32),
            pltpu.VMEM((BPW,), jnp.float32),
            pltpu.SemaphoreType.DMA,
        ],
    )
    return f(nodes32, emb_table, wb, ewf)


# speedup vs baseline: 1.6532x; 1.6532x over previous
import functools

import jax
import jax.numpy as jnp
from jax import lax
from jax.experimental import pallas as pl
from jax.experimental.pallas import tpu as pltpu
from jax.experimental.pallas import tpu_sc as plsc

N = 16384
V = 1000000
D = 64
E = 4

NC = 2
NS = 16
L = 16
NW = NC * NS
BPW = N // NW
GPB = 8
BLK = GPB * L
NBLK = BPW // BLK
WAVE = 64
NWAVE = BPW // WAVE


def _sc_body(nodes_hbm, emb_hbm, wb_hbm, ew_hbm, out_hbm,
             idx_v, idx_s, rows_v, wb_v, ew_v, sco_v, res_v, sem):
    wid = lax.axis_index("s") * NC + lax.axis_index("c")
    base = wid * BPW

    pltpu.sync_copy(nodes_hbm.at[pl.ds(base, BPW)], idx_v)
    pltpu.sync_copy(wb_hbm, wb_v)
    pltpu.sync_copy(ew_hbm, ew_v)

    def issue_wave(w):
        def issue(j, _):
            i0 = w * WAVE + j * L
            vec = idx_v[pl.ds(i0, L)]
            for k in range(L):
                pltpu.async_copy(emb_hbm.at[pl.ds(vec[k], 1), :],
                                 rows_v.at[pl.ds(i0 + k, 1), :], sem)
            return 0
        lax.fori_loop(0, WAVE // L, issue, 0)

    def drain_wave(w):
        def drain(j, _):
            i = w * WAVE + j
            pltpu.make_async_copy(emb_hbm.at[pl.ds(0, 1), :],
                                  rows_v.at[pl.ds(i, 1), :], sem).wait()
            return 0
        lax.fori_loop(0, WAVE, drain, 0)

    issue_wave(0)

    def wave_body(w, _):
        issue_wave(w)
        drain_wave(w - 1)
        return 0

    lax.fori_loop(1, NWAVE, wave_body, 0)
    drain_wave(NWAVE - 1)

    lanes = lax.iota(jnp.int32, L)
    zero = jnp.zeros((L,), jnp.float32)

    def block_body(b, carry):
        tok0 = b * BLK
        toks = [tok0 + g * L + lanes for g in range(GPB)]

        def d_body(d, accs):
            w = [wb_v[pl.ds((e * D) * L + d * L, L)] for e in range(E)]
            dvec = jnp.full((L,), 0, jnp.int32) + d
            new = []
            for g in range(GPB):
                col = plsc.load_gather(rows_v, [toks[g], dvec])
                prev = accs[g]
                new.append(tuple(prev[e] + col * w[e] for e in range(E)))
            return tuple(new)

        init = tuple(tuple(zero for _ in range(E)) for _ in range(GPB))
        accs = lax.fori_loop(0, D, d_body, init)

        for g in range(GPB):
            s0, s1, s2, s3 = accs[g]
            m = jnp.maximum(jnp.maximum(s0, s1), jnp.maximum(s2, s3))
            e0 = jnp.exp(s0 - m)
            e1 = jnp.exp(s1 - m)
            e2 = jnp.exp(s2 - m)
            e3 = jnp.exp(s3 - m)
            r = 1.0 / (e0 + e1 + e2 + e3)
            p0, p1, p2, p3 = e0 * r, e1 * r, e2 * r, e3 * r
            c0 = (s0 >= s1) & (s0 >= s2) & (s0 >= s3)
            c1 = (s1 >= s2) & (s1 >= s3)
            c2 = s2 >= s3
            two = jnp.full((L,), 2, jnp.int32)
            three = jnp.full((L,), 3, jnp.int32)
            one = jnp.full((L,), 1, jnp.int32)
            zi = jnp.full((L,), 0, jnp.int32)
            choice = jnp.where(c0, zi, jnp.where(c1, one, jnp.where(c2, two, three)))
            c4 = choice * 4
            ps = (p0, p1, p2, p3)
            res = zero
            for j in range(E):
                wj = plsc.load_gather(ew_v, [c4 + jnp.full((L,), j, jnp.int32)])
                res = res + wj * ps[j]
            off = tok0 + g * L
            sco_v[pl.ds(0 * BPW + off, L)] = p0
            sco_v[pl.ds(1 * BPW + off, L)] = p1
            sco_v[pl.ds(2 * BPW + off, L)] = p2
            sco_v[pl.ds(3 * BPW + off, L)] = p3
            res_v[pl.ds(off, L)] = res
        return carry

    lax.fori_loop(0, NBLK, block_body, 0)

    for e in range(E):
        pltpu.sync_copy(sco_v.at[pl.ds(e * BPW, BPW)],
                        out_hbm.at[pl.ds(e * N + base, BPW)])
    pltpu.sync_copy(res_v, out_hbm.at[pl.ds(E * N + base, BPW)])


@jax.jit
def kernel(nodes, emb_table, W, expert_w):
    wb = jnp.broadcast_to(W[:, :, None], (E, D, L)).reshape(-1).astype(jnp.float32)
    ewf = expert_w.reshape(-1).astype(jnp.float32)
    nodes32 = nodes.astype(jnp.int32)
    mesh = plsc.VectorSubcoreMesh(core_axis_name="c", subcore_axis_name="s")
    f = pl.kernel(
        _sc_body,
        out_type=jax.ShapeDtypeStruct((E * N + N,), jnp.float32),
        mesh=mesh,
        compiler_params=pltpu.CompilerParams(needs_layout_passes=False),
        scratch_types=[
            pltpu.VMEM((BPW,), jnp.int32),
            pltpu.SMEM((BPW,), jnp.int32),
            pltpu.VMEM((BPW, D), jnp.float32),
            pltpu.VMEM((E * D * L,), jnp.float32),
            pltpu.VMEM((E * E,), jnp.float32),
            pltpu.VMEM((E * BPW,), jnp.float32),
            pltpu.VMEM((BPW,), jnp.float32),
            pltpu.SemaphoreType.DMA,
        ],
    )
    return f(nodes32, emb_table, wb, ewf)
